# baseline (speedup 1.0000x reference)
import jax
import jax.numpy as jnp
from jax.experimental import pallas as pl
from jax.experimental.pallas import tpu as pltpu

_NUM_CLASSES = 30522
_ROWS = 1024
_BR = 32
_NBUF = 4
_NBLK = _ROWS // _BR


def _onehot_block(x_ref, o_ref, buf, sems):
    i = pl.program_id(0)
    slot = jax.lax.rem(i, _NBUF)

    def _copy(s, blk):
        return pltpu.make_async_copy(
            buf.at[s], o_ref.at[pl.ds(blk * _BR, _BR), :], sems.at[s]
        )

    @pl.when(i >= _NBUF)
    def _():
        _copy(slot, i).wait()

    cols = jax.lax.broadcasted_iota(jnp.int32, (_BR, _NUM_CLASSES), 1)
    buf[slot] = (x_ref[pl.ds(i * _BR, _BR), :] == cols).astype(jnp.int32)
    _copy(slot, i).start()

    @pl.when(i == _NBLK - 1)
    def _():
        for s in range(_NBUF):
            _copy(s, i).wait()


def kernel(x):
    x2 = x.reshape(_ROWS, 1)
    return pl.pallas_call(
        _onehot_block,
        grid=(_NBLK,),
        in_specs=[pl.BlockSpec((_ROWS, 1), lambda i: (0, 0))],
        out_specs=pl.BlockSpec(memory_space=pl.ANY),
        out_shape=jax.ShapeDtypeStruct((_ROWS, _NUM_CLASSES), jnp.int32),
        scratch_shapes=[
            pltpu.VMEM((_NBUF, _BR, _NUM_CLASSES), jnp.int32),
            pltpu.SemaphoreType.DMA((_NBUF,)),
        ],
        compiler_params=pltpu.CompilerParams(
            dimension_semantics=("arbitrary",),
        ),
    )(x2)


# speedup vs baseline: 3.3245x; 3.3245x over previous
import jax
import jax.numpy as jnp
from jax.experimental import pallas as pl
from jax.experimental.pallas import tpu as pltpu

_NUM_CLASSES = 30522
_ROWS = 1024
_BC = 512


def _onehot_block(x_ref, t_ref):
    i = pl.program_id(0)
    classes = jax.lax.broadcasted_iota(jnp.int32, (_BC, _ROWS), 0) + i * _BC
    t_ref[...] = (x_ref[...] == classes).astype(jnp.int32)


def kernel(x):
    x2 = x.reshape(1, _ROWS)
    t = pl.pallas_call(
        _onehot_block,
        grid=(pl.cdiv(_NUM_CLASSES, _BC),),
        in_specs=[pl.BlockSpec((1, _ROWS), lambda i: (0, 0))],
        out_specs=pl.BlockSpec((_BC, _ROWS), lambda i: (i, 0)),
        out_shape=jax.ShapeDtypeStruct((_NUM_CLASSES, _ROWS), jnp.int32),
        compiler_params=pltpu.CompilerParams(
            dimension_semantics=("parallel",),
        ),
    )(x2)
    return t.T


# speedup vs baseline: 3.9269x; 1.1812x over previous
import jax
import jax.numpy as jnp
from jax.experimental import pallas as pl
from jax.experimental.pallas import tpu as pltpu

_NUM_CLASSES = 30522
_ROWS = 1024
_BC = 1024


def _onehot_block(x_ref, t_ref):
    i = pl.program_id(0)
    classes = jax.lax.broadcasted_iota(jnp.int32, (_BC, _ROWS), 0) + i * _BC
    t_ref[...] = (x_ref[...] == classes).astype(jnp.int32)


def kernel(x):
    x2 = x.reshape(1, _ROWS)
    t = pl.pallas_call(
        _onehot_block,
        grid=(pl.cdiv(_NUM_CLASSES, _BC),),
        in_specs=[pl.BlockSpec((1, _ROWS), lambda i: (0, 0))],
        out_specs=pl.BlockSpec((_BC, _ROWS), lambda i: (i, 0)),
        out_shape=jax.ShapeDtypeStruct((_NUM_CLASSES, _ROWS), jnp.int32),
        compiler_params=pltpu.CompilerParams(
            dimension_semantics=("parallel",),
        ),
    )(x2)
    return t.T
